# Initial kernel scaffold; baseline (speedup 1.0000x reference)
import jax
import jax.numpy as jnp
from jax.experimental import pallas as pl


def kernel(primary, ss, x, y, primary_table, ss_table, W_cord, b_cord, gamma, beta):
    raise NotImplementedError("write your pallas kernel here")


# speedup vs baseline: 1.2667x; 1.2667x over previous
import functools

import jax
import jax.numpy as jnp
from jax import lax
from jax.experimental import pallas as pl
from jax.experimental.pallas import tpu as pltpu
from jax.experimental.pallas import tpu_sc as plsc

_B, _L, _V, _D = 4, 4096, 1000, 2048
_N = _B * _L
_NC, _NS = 2, 16
_NW = _NC * _NS
_PER_W = _N // _NW
_C = 16
_G = _PER_W // _C
_LANES = 16


def _make_gather_sum():
    mesh = plsc.VectorSubcoreMesh(
        core_axis_name="c", subcore_axis_name="s",
        num_cores=_NC, num_subcores=_NS)

    @functools.partial(
        pl.kernel,
        out_type=jax.ShapeDtypeStruct((_N, _D), jnp.float32),
        mesh=mesh,
        scratch_types=[
            pltpu.VMEM((_C,), jnp.int32),
            pltpu.VMEM((_C,), jnp.int32),
            pltpu.VMEM((_C, _D), jnp.float32),
            pltpu.VMEM((_C, _D), jnp.float32),
            pltpu.SemaphoreType.DMA,
            pltpu.SemaphoreType.DMA,
        ],
    )
    def gather_sum(pidx_hbm, sidx_hbm, ptab_hbm, stab_hbm, out_hbm,
                   idxp_v, idxs_v, bufp_v, bufs_v, semp, sems):
        wid = lax.axis_index("s") * _NC + lax.axis_index("c")

        def chunk(g, carry):
            base = wid * _PER_W + g * _C
            pltpu.sync_copy(pidx_hbm.at[pl.ds(base, _C)], idxp_v)
            pltpu.sync_copy(sidx_hbm.at[pl.ds(base, _C)], idxs_v)
            cp = pltpu.async_copy(ptab_hbm.at[idxp_v], bufp_v, semp)
            cs = pltpu.async_copy(stab_hbm.at[idxs_v], bufs_v, sems)
            cp.wait()
            cs.wait()

            def add_row(c, carry2):
                def add_vec(j, carry3):
                    o = j * _LANES
                    bufp_v[c, pl.ds(o, _LANES)] = (
                        bufp_v[c, pl.ds(o, _LANES)] + bufs_v[c, pl.ds(o, _LANES)])
                    return carry3
                return lax.fori_loop(0, _D // _LANES, add_vec, carry2)

            lax.fori_loop(0, _C, add_row, 0)
            pltpu.sync_copy(bufp_v, out_hbm.at[pl.ds(base, _C)])
            return carry

        lax.fori_loop(0, _G, chunk, 0)

    return gather_sum


_gather_sum = _make_gather_sum()


def _ln_body(e_ref, x_ref, y_ref, w0_ref, wy_ref, b_ref, g_ref, bt_ref, o_ref):
    e = e_ref[...]
    cord = x_ref[...] * w0_ref[...] + y_ref[...] * wy_ref[...] + b_ref[...]
    e = e + cord
    mean = jnp.mean(e, axis=1, keepdims=True)
    d = e - mean
    var = jnp.mean(d * d, axis=1, keepdims=True)
    o_ref[...] = d * lax.rsqrt(var + 1e-5) * g_ref[...] + bt_ref[...]


_T = 512


def _ln_call(esum, x2, y2, w0, wy, b2, g2, bt2):
    vec = pl.BlockSpec((1, _D), lambda i: (0, 0))
    return pl.pallas_call(
        _ln_body,
        grid=(_N // _T,),
        in_specs=[
            pl.BlockSpec((_T, _D), lambda i: (i, 0)),
            pl.BlockSpec((_T, 1), lambda i: (i, 0)),
            pl.BlockSpec((_T, 1), lambda i: (i, 0)),
            vec, vec, vec, vec, vec,
        ],
        out_specs=pl.BlockSpec((_T, _D), lambda i: (i, 0)),
        out_shape=jax.ShapeDtypeStruct((_N, _D), jnp.float32),
    )(esum, x2, y2, w0, wy, b2, g2, bt2)


def kernel(primary, ss, x, y, primary_table, ss_table, W_cord, b_cord, gamma, beta):
    pidx = primary.reshape(_N).astype(jnp.int32)
    sidx = ss.reshape(_N).astype(jnp.int32)
    esum = _gather_sum(pidx, sidx, primary_table, ss_table)
    w0 = W_cord[0:1]
    wy = (W_cord[1] + W_cord[2])[None]
    out = _ln_call(esum, x.reshape(_N, 1), y.reshape(_N, 1),
                   w0, wy, b_cord[None], gamma[None], beta[None])
    return out.reshape(_B, _L, _D)


# speedup vs baseline: 2.8135x; 2.2210x over previous
import functools

import jax
import jax.numpy as jnp
from jax import lax
from jax.experimental import pallas as pl
from jax.experimental.pallas import tpu as pltpu
from jax.experimental.pallas import tpu_sc as plsc

_B, _L, _V, _D = 4, 4096, 1000, 2048
_N = _B * _L
_NC, _NS = 2, 16
_NW = _NC * _NS
_PER_W = _N // _NW
_C = 8
_G = _PER_W // _C
_LANES = 16


def _make_gather_sum():
    mesh = plsc.VectorSubcoreMesh(
        core_axis_name="c", subcore_axis_name="s",
        num_cores=_NC, num_subcores=_NS)

    @functools.partial(
        pl.kernel,
        out_type=jax.ShapeDtypeStruct((_N, _D), jnp.float32),
        mesh=mesh,
        scratch_types=[
            pltpu.VMEM((_PER_W,), jnp.int32),
            pltpu.VMEM((_PER_W,), jnp.int32),
            pltpu.VMEM((_C, _D), jnp.float32),
            pltpu.VMEM((_C, _D), jnp.float32),
            pltpu.VMEM((_C, _D), jnp.float32),
            pltpu.VMEM((_C, _D), jnp.float32),
            pltpu.VMEM((_C, _D), jnp.float32),
            pltpu.VMEM((_C, _D), jnp.float32),
            pltpu.SemaphoreType.DMA,
            pltpu.SemaphoreType.DMA,
            pltpu.SemaphoreType.DMA,
            pltpu.SemaphoreType.DMA,
            pltpu.SemaphoreType.DMA,
            pltpu.SemaphoreType.DMA,
        ],
    )
    def gather_sum(pidx_hbm, sidx_hbm, ptab_hbm, stab_hbm, out_hbm,
                   idxp_v, idxs_v, bufp0, bufp1, bufs0, bufs1, bufo0, bufo1,
                   semp0, semp1, sems0, sems1, semw0, semw1):
        bufp = (bufp0, bufp1)
        bufs = (bufs0, bufs1)
        bufo = (bufo0, bufo1)
        semp = (semp0, semp1)
        sems = (sems0, sems1)
        semw = (semw0, semw1)

        wid = lax.axis_index("s") * _NC + lax.axis_index("c")
        wbase = wid * _PER_W
        pltpu.sync_copy(pidx_hbm.at[pl.ds(wbase, _PER_W)], idxp_v)
        pltpu.sync_copy(sidx_hbm.at[pl.ds(wbase, _PER_W)], idxs_v)

        def gather_pair(g, b):
            pltpu.async_copy(
                ptab_hbm.at[idxp_v.at[pl.ds(g * _C, _C)]], bufp[b], semp[b])
            pltpu.async_copy(
                stab_hbm.at[idxs_v.at[pl.ds(g * _C, _C)]], bufs[b], sems[b])

        def wait_gather_pair(g, b):
            pltpu.make_async_copy(
                ptab_hbm.at[idxp_v.at[pl.ds(g * _C, _C)]], bufp[b], semp[b]).wait()
            pltpu.make_async_copy(
                stab_hbm.at[idxs_v.at[pl.ds(g * _C, _C)]], bufs[b], sems[b]).wait()

        gather_pair(0, 0)
        gather_pair(1, 1)

        def add_chunk(b):
            def row(c, carry):
                for j in range(_D // _LANES):
                    o = j * _LANES
                    bufo[b][c, pl.ds(o, _LANES)] = (
                        bufp[b][c, pl.ds(o, _LANES)]
                        + bufs[b][c, pl.ds(o, _LANES)])
                return carry
            lax.fori_loop(0, _C, row, 0)

        def outer(g2, carry):
            for b in (0, 1):
                g = g2 * 2 + b
                base = wbase + g * _C
                wait_gather_pair(g, b)
                @pl.when(g >= 2)
                def _():
                    pltpu.make_async_copy(
                        bufo[b], out_hbm.at[pl.ds(base, _C)], semw[b]).wait()
                add_chunk(b)
                pltpu.async_copy(bufo[b], out_hbm.at[pl.ds(base, _C)], semw[b])
                @pl.when(g + 2 < _G)
                def _():
                    gather_pair(g + 2, b)
            return carry

        lax.fori_loop(0, _G // 2, outer, 0)

        for b in (0, 1):
            pltpu.make_async_copy(
                bufo[b], out_hbm.at[pl.ds(wbase, _C)], semw[b]).wait()

    return gather_sum


_gather_sum = _make_gather_sum()


def _ln_body(e_ref, x_ref, y_ref, w0_ref, wy_ref, b_ref, g_ref, bt_ref, o_ref):
    e = e_ref[...]
    cord = x_ref[...] * w0_ref[...] + y_ref[...] * wy_ref[...] + b_ref[...]
    e = e + cord
    mean = jnp.mean(e, axis=1, keepdims=True)
    d = e - mean
    var = jnp.mean(d * d, axis=1, keepdims=True)
    o_ref[...] = d * lax.rsqrt(var + 1e-5) * g_ref[...] + bt_ref[...]


_T = 512


def _ln_call(esum, x2, y2, w0, wy, b2, g2, bt2):
    vec = pl.BlockSpec((1, _D), lambda i: (0, 0))
    return pl.pallas_call(
        _ln_body,
        grid=(_N // _T,),
        in_specs=[
            pl.BlockSpec((_T, _D), lambda i: (i, 0)),
            pl.BlockSpec((_T, 1), lambda i: (i, 0)),
            pl.BlockSpec((_T, 1), lambda i: (i, 0)),
            vec, vec, vec, vec, vec,
        ],
        out_specs=pl.BlockSpec((_T, _D), lambda i: (i, 0)),
        out_shape=jax.ShapeDtypeStruct((_N, _D), jnp.float32),
    )(esum, x2, y2, w0, wy, b2, g2, bt2)


def kernel(primary, ss, x, y, primary_table, ss_table, W_cord, b_cord, gamma, beta):
    pidx = primary.reshape(_N).astype(jnp.int32)
    sidx = ss.reshape(_N).astype(jnp.int32)
    esum = _gather_sum(pidx, sidx, primary_table, ss_table)
    w0 = W_cord[0:1]
    wy = (W_cord[1] + W_cord[2])[None]
    out = _ln_call(esum, x.reshape(_N, 1), y.reshape(_N, 1),
                   w0, wy, b_cord[None], gamma[None], beta[None])
    return out.reshape(_B, _L, _D)
